# baseline (speedup 1.0000x reference)
import functools

import jax
import jax.numpy as jnp
from jax import lax
from jax.experimental import pallas as pl
from jax.experimental.pallas import tpu as pltpu
from jax.experimental.pallas import tpu_sc as plsc

_N = 320000
_D = 128
_S = 1024
_C = 128
_NCHUNK = _N // _C
_NC = 2
_NS = 16
_NW = _NC * _NS
_CPW = _NCHUNK // _NW
_XTRA = _NCHUNK - _CPW * _NW
_NGRP = _CPW // 6
_RPT = _S // _NS


def _sc_segment_sum(x1, x2, x3, batch, scalesb):
    mesh = plsc.VectorSubcoreMesh(core_axis_name="c", subcore_axis_name="s")

    @functools.partial(
        pl.kernel,
        mesh=mesh,
        out_type=jax.ShapeDtypeStruct((_NC, _S, _D), jnp.float32),
        scratch_types=[
            pltpu.VMEM((_C, _D), jnp.float32),
            pltpu.VMEM((_C, _D), jnp.float32),
            pltpu.VMEM((_C, _D), jnp.float32),
            pltpu.VMEM((_C, _D), jnp.float32),
            pltpu.VMEM((_C, _D), jnp.float32),
            pltpu.VMEM((_C, _D), jnp.float32),
            pltpu.VMEM((_C, _D), jnp.float32),
            pltpu.VMEM((_C,), jnp.int32),
            pltpu.VMEM((_C,), jnp.int32),
            pltpu.VMEM((_C,), jnp.int32),
            pltpu.VMEM((3, 16), jnp.float32),
            pltpu.VMEM_SHARED((_S, _D), jnp.float32),
            pltpu.SemaphoreType.DMA,
            pltpu.SemaphoreType.DMA,
            pltpu.SemaphoreType.DMA,
            pltpu.SemaphoreType.DMA,
            pltpu.SemaphoreType.DMA,
            pltpu.SemaphoreType.DMA,
            pltpu.SemaphoreType.DMA,
            pltpu.SemaphoreType.DMA,
        ],
    )
    def body(x1h, x2h, x3h, bh, sclh, outh,
             x1b0, x1b1, x2b0, x2b1, x3b0, x3b1, x3b2, ix0, ix1, ix2,
             scl_v, acc, l12s0, l12s1, l3s0, l3s1, l3s2, ss0, ss1, ss2):
        cid = lax.axis_index("c")
        sid = lax.axis_index("s")
        wid = sid * _NC + cid
        x1b = (x1b0, x1b1)
        x2b = (x2b0, x2b1)
        x3b = (x3b0, x3b1, x3b2)
        ixb = (ix0, ix1, ix2)
        l12s = (l12s0, l12s1)
        l3s = (l3s0, l3s1, l3s2)
        ss = (ss0, ss1, ss2)

        def rowbase(c):
            return jnp.minimum(c, _NCHUNK - 1) * _C

        def load12(c, j2):
            base = rowbase(c)
            pltpu.async_copy(x1h.at[pl.ds(base, _C)], x1b[j2], l12s[j2])
            pltpu.async_copy(x2h.at[pl.ds(base, _C)], x2b[j2], l12s[j2])

        def drain12(j2):
            pltpu.make_async_copy(x1h.at[pl.ds(0, _C)], x1b[j2],
                                  l12s[j2]).wait()
            pltpu.make_async_copy(x2h.at[pl.ds(0, _C)], x2b[j2],
                                  l12s[j2]).wait()

        def load3(c, j3):
            base = rowbase(c)
            pltpu.async_copy(x3h.at[pl.ds(base, _C)], x3b[j3], l3s[j3])
            pltpu.async_copy(bh.at[pl.ds(base, _C)], ixb[j3], l3s[j3])

        def drain3(j3):
            pltpu.make_async_copy(x3h.at[pl.ds(0, _C)], x3b[j3],
                                  l3s[j3]).wait()
            pltpu.make_async_copy(bh.at[pl.ds(0, _C)], ixb[j3],
                                  l3s[j3]).wait()

        def combine(j2, j3):
            r1, r2, r3 = x1b[j2], x2b[j2], x3b[j3]
            s1 = scl_v[0]
            s2 = scl_v[1]
            s3 = scl_v[2]

            def row_body(r, carry):
                for j in range(_D // 16):
                    sl = pl.ds(j * 16, 16)
                    r3[r, sl] = (r1[r, sl] * s1 + r2[r, sl] * s2
                                 + r3[r, sl] * s3)
                return carry

            lax.fori_loop(0, _C, row_body, 0)

        def scat(j3):
            pltpu.async_copy(x3b[j3], acc.at[ixb[j3]], ss[j3], add=True)

        def drain_scat(j3):
            pltpu.make_async_copy(x3b[j3], acc.at[pl.ds(0, _C)],
                                  ss[j3]).wait()

        def zrow_body(r, carry):
            for j in range(_D // 16):
                x1b0[r, pl.ds(j * 16, 16)] = jnp.zeros((16,), jnp.float32)
            return carry

        lax.fori_loop(0, _RPT, zrow_body, 0)
        pltpu.sync_copy(x1b0.at[pl.ds(0, _RPT)],
                        acc.at[pl.ds(sid * _RPT, _RPT)])
        pltpu.sync_copy(sclh, scl_v)
        plsc.subcore_barrier()

        s_w = wid * _CPW + jnp.minimum(wid, _XTRA)

        load12(s_w, 0)
        load12(s_w + 1, 1)
        load3(s_w, 0)
        load3(s_w + 1, 1)

        def group(g, carry):
            c = s_w + 6 * g
            for i in range(6):
                j2 = i % 2
                j3 = i % 3
                jn = (i + 1) % 3
                drain12(j2)
                drain3(j3)
                combine(j2, j3)
                scat(j3)
                load12(c + i + 2, j2)
                if i == 0:
                    @pl.when(g > 0)
                    def _():
                        drain_scat(jn)
                        load3(c + i + 1, jn)
                elif i == 1:
                    @pl.when(g > 0)
                    def _():
                        drain_scat(jn)

                    load3(c + i + 1, jn)
                else:
                    drain_scat(jn)
                    load3(c + i + 1, jn)
            return carry

        lax.fori_loop(0, _NGRP, group, 0)

        drain12(0)
        drain3(0)

        @pl.when(wid < _XTRA)
        def _():
            combine(0, 0)
            scat(0)

        drain_scat(1)
        drain_scat(2)

        @pl.when(wid < _XTRA)
        def _():
            drain_scat(0)

        drain12(1)
        plsc.subcore_barrier()

        r0 = sid * _RPT
        pltpu.sync_copy(acc.at[pl.ds(r0, _RPT)], x1b0.at[pl.ds(0, _RPT)])
        pltpu.sync_copy(x1b0.at[pl.ds(0, _RPT)],
                        outh.at[cid].at[pl.ds(r0, _RPT)])

    return body(x1, x2, x3, batch, scalesb)


def _tc_add(partials):
    def body(p_ref, o_ref):
        o_ref[...] = p_ref[0] + p_ref[1]

    return pl.pallas_call(
        body,
        out_shape=jax.ShapeDtypeStruct((_S, _D), jnp.float32),
    )(partials)


def kernel(x1, x2, x3, batch, scales):
    batch_i = batch.astype(jnp.int32)
    scalesb = jnp.broadcast_to(
        scales.astype(jnp.float32)[:, None], (3, 16))
    partials = _sc_segment_sum(x1, x2, x3, batch_i, scalesb)
    return _tc_add(partials)


# speedup vs baseline: 1.0002x; 1.0002x over previous
import functools

import jax
import jax.numpy as jnp
from jax import lax
from jax.experimental import pallas as pl
from jax.experimental.pallas import tpu as pltpu
from jax.experimental.pallas import tpu_sc as plsc

_N = 320000
_D = 128
_S = 1024
_C = 128
_NCHUNK = _N // _C
_NC = 2
_NS = 16
_NW = _NC * _NS
_CPW = _NCHUNK // _NW
_XTRA = _NCHUNK - _CPW * _NW
_NGRP = _CPW // 6
_RPT = _S // _NS


def _sc_segment_sum(x1, x2, x3, batch, scalesb):
    mesh = plsc.VectorSubcoreMesh(core_axis_name="c", subcore_axis_name="s")

    @functools.partial(
        pl.kernel,
        mesh=mesh,
        out_type=jax.ShapeDtypeStruct((_NC, _S, _D), jnp.float32),
        scratch_types=[
            pltpu.VMEM((_C, _D), jnp.float32),
            pltpu.VMEM((_C, _D), jnp.float32),
            pltpu.VMEM((_C, _D), jnp.float32),
            pltpu.VMEM((_C, _D), jnp.float32),
            pltpu.VMEM((_C, _D), jnp.float32),
            pltpu.VMEM((_C, _D), jnp.float32),
            pltpu.VMEM((_C, _D), jnp.float32),
            pltpu.VMEM((_C,), jnp.int32),
            pltpu.VMEM((_C,), jnp.int32),
            pltpu.VMEM((_C,), jnp.int32),
            pltpu.VMEM((3, 16), jnp.float32),
            pltpu.VMEM_SHARED((_S, _D), jnp.float32),
            pltpu.SemaphoreType.DMA,
            pltpu.SemaphoreType.DMA,
            pltpu.SemaphoreType.DMA,
            pltpu.SemaphoreType.DMA,
            pltpu.SemaphoreType.DMA,
            pltpu.SemaphoreType.DMA,
            pltpu.SemaphoreType.DMA,
            pltpu.SemaphoreType.DMA,
        ],
    )
    def body(x1h, x2h, x3h, bh, sclh, outh,
             x1b0, x1b1, x2b0, x2b1, x3b0, x3b1, x3b2, ix0, ix1, ix2,
             scl_v, acc, l12s0, l12s1, l3s0, l3s1, l3s2, ss0, ss1, ss2):
        cid = lax.axis_index("c")
        sid = lax.axis_index("s")
        wid = sid * _NC + cid
        x1b = (x1b0, x1b1)
        x2b = (x2b0, x2b1)
        x3b = (x3b0, x3b1, x3b2)
        ixb = (ix0, ix1, ix2)
        l12s = (l12s0, l12s1)
        l3s = (l3s0, l3s1, l3s2)
        ss = (ss0, ss1, ss2)

        def rowbase(c):
            return jnp.minimum(c, _NCHUNK - 1) * _C

        def load12(c, j2):
            base = rowbase(c)
            pltpu.async_copy(x1h.at[pl.ds(base, _C)], x1b[j2], l12s[j2])
            pltpu.async_copy(x2h.at[pl.ds(base, _C)], x2b[j2], l12s[j2])

        def drain12(j2):
            pltpu.make_async_copy(x1h.at[pl.ds(0, _C)], x1b[j2],
                                  l12s[j2]).wait()
            pltpu.make_async_copy(x2h.at[pl.ds(0, _C)], x2b[j2],
                                  l12s[j2]).wait()

        def load3(c, j3):
            base = rowbase(c)
            pltpu.async_copy(x3h.at[pl.ds(base, _C)], x3b[j3], l3s[j3])
            pltpu.async_copy(bh.at[pl.ds(base, _C)], ixb[j3], l3s[j3])

        def drain3(j3):
            pltpu.make_async_copy(x3h.at[pl.ds(0, _C)], x3b[j3],
                                  l3s[j3]).wait()
            pltpu.make_async_copy(bh.at[pl.ds(0, _C)], ixb[j3],
                                  l3s[j3]).wait()

        def combine(j2, j3):
            r1, r2, r3 = x1b[j2], x2b[j2], x3b[j3]
            s1 = scl_v[0]
            s2 = scl_v[1]
            s3 = scl_v[2]

            def row_body(r, carry):
                for j in range(_D // 16):
                    sl = pl.ds(j * 16, 16)
                    r3[r, sl] = (r1[r, sl] * s1 + r2[r, sl] * s2
                                 + r3[r, sl] * s3)
                return carry

            lax.fori_loop(0, _C, row_body, 0)

        def scat(j3):
            pltpu.async_copy(x3b[j3], acc.at[ixb[j3]], ss[j3], add=True)

        def drain_scat(j3):
            pltpu.make_async_copy(x3b[j3], acc.at[pl.ds(0, _C)],
                                  ss[j3]).wait()

        def zrow_body(r, carry):
            for j in range(_D // 16):
                x1b0[r, pl.ds(j * 16, 16)] = jnp.zeros((16,), jnp.float32)
            return carry

        lax.fori_loop(0, _RPT, zrow_body, 0)
        pltpu.sync_copy(x1b0.at[pl.ds(0, _RPT)],
                        acc.at[pl.ds(sid * _RPT, _RPT)])
        pltpu.sync_copy(sclh, scl_v)
        plsc.subcore_barrier()

        s_w = wid * _CPW + jnp.minimum(wid, _XTRA)

        load12(s_w, 0)
        load3(s_w, 0)
        load12(s_w + 1, 1)
        load3(s_w + 1, 1)

        def group(g, carry):
            c = s_w + 6 * g
            for i in range(6):
                j2 = i % 2
                j3 = i % 3
                jn = (i + 1) % 3
                drain12(j2)
                drain3(j3)
                combine(j2, j3)
                if i == 0:
                    @pl.when(g > 0)
                    def _():
                        drain_scat(jn)
                        load3(c + i + 1, jn)
                elif i == 1:
                    @pl.when(g > 0)
                    def _():
                        drain_scat(jn)

                    load3(c + i + 1, jn)
                else:
                    drain_scat(jn)
                    load3(c + i + 1, jn)
                scat(j3)
                load12(c + i + 2, j2)
            return carry

        lax.fori_loop(0, _NGRP, group, 0)

        drain12(0)
        drain3(0)

        @pl.when(wid < _XTRA)
        def _():
            combine(0, 0)
            scat(0)

        drain_scat(1)
        drain_scat(2)

        @pl.when(wid < _XTRA)
        def _():
            drain_scat(0)

        drain12(1)
        plsc.subcore_barrier()

        r0 = sid * _RPT
        pltpu.sync_copy(acc.at[pl.ds(r0, _RPT)], x1b0.at[pl.ds(0, _RPT)])
        pltpu.sync_copy(x1b0.at[pl.ds(0, _RPT)],
                        outh.at[cid].at[pl.ds(r0, _RPT)])

    return body(x1, x2, x3, batch, scalesb)


def _tc_add(partials):
    def body(p_ref, o_ref):
        o_ref[...] = p_ref[0] + p_ref[1]

    return pl.pallas_call(
        body,
        out_shape=jax.ShapeDtypeStruct((_S, _D), jnp.float32),
    )(partials)


def kernel(x1, x2, x3, batch, scales):
    batch_i = batch.astype(jnp.int32)
    scalesb = jnp.broadcast_to(
        scales.astype(jnp.float32)[:, None], (3, 16))
    partials = _sc_segment_sum(x1, x2, x3, batch_i, scalesb)
    return _tc_add(partials)


# speedup vs baseline: 1.3507x; 1.3504x over previous
import functools

import jax
import jax.numpy as jnp
from jax import lax
from jax.experimental import pallas as pl
from jax.experimental.pallas import tpu as pltpu
from jax.experimental.pallas import tpu_sc as plsc

_N = 320000
_D = 128
_S = 1024
_C = 128
_NCHUNK = _N // _C
_NC = 2
_NS = 16
_NW = _NC * _NS
_CPW = _NCHUNK // _NW
_XTRA = _NCHUNK - _CPW * _NW
_NPAIR = _CPW // 2
_IPW = _CPW + 1 + 9
_RPT = _S // _NS


def _sc_segment_sum(x1, x2, x3, batch, scalesb):
    mesh = plsc.VectorSubcoreMesh(core_axis_name="c", subcore_axis_name="s")

    @functools.partial(
        pl.kernel,
        mesh=mesh,
        out_type=jax.ShapeDtypeStruct((_NC, _S, _D), jnp.float32),
        scratch_types=[
            pltpu.VMEM((_C, _D), jnp.float32),
            pltpu.VMEM((_C, _D), jnp.float32),
            pltpu.VMEM((_C, _D), jnp.float32),
            pltpu.VMEM((_C, _D), jnp.float32),
            pltpu.VMEM((_C, _D), jnp.float32),
            pltpu.VMEM((_C, _D), jnp.float32),
            pltpu.VMEM((_IPW, _C), jnp.int32),
            pltpu.VMEM((3, 16), jnp.float32),
            pltpu.VMEM_SHARED((_S, _D), jnp.float32),
            pltpu.SemaphoreType.DMA,
            pltpu.SemaphoreType.DMA,
            pltpu.SemaphoreType.DMA,
            pltpu.SemaphoreType.DMA,
        ],
    )
    def body(x1h, x2h, x3h, bh, sclh, outh,
             r1a, r2a, r3a, r1b, r2b, r3b, idx_v, scl_v,
             acc, lsa, lsb, ssa, ssb):
        cid = lax.axis_index("c")
        sid = lax.axis_index("s")
        wid = sid * _NC + cid
        bufs_a = (r1a, r2a, r3a)
        bufs_b = (r1b, r2b, r3b)

        def issue_loads(c, bufs, sem):
            base = c * _C
            r1, r2, r3 = bufs
            pltpu.async_copy(x1h.at[pl.ds(base, _C)], r1, sem)
            pltpu.async_copy(x2h.at[pl.ds(base, _C)], r2, sem)
            pltpu.async_copy(x3h.at[pl.ds(base, _C)], r3, sem)

        def drain_loads(bufs, sem):
            r1, r2, r3 = bufs
            pltpu.make_async_copy(x1h.at[pl.ds(0, _C)], r1, sem).wait()
            pltpu.make_async_copy(x2h.at[pl.ds(0, _C)], r2, sem).wait()
            pltpu.make_async_copy(x3h.at[pl.ds(0, _C)], r3, sem).wait()

        def combine(bufs):
            r1, r2, r3 = bufs
            s1 = scl_v[0]
            s2 = scl_v[1]
            s3 = scl_v[2]

            def row_body(r, carry):
                for j in range(_D // 16):
                    sl = pl.ds(j * 16, 16)
                    r1[r, sl] = (r1[r, sl] * s1 + r2[r, sl] * s2
                                 + r3[r, sl] * s3)
                return carry

            lax.fori_loop(0, _C, row_body, 0)

        def issue_scat(k, bufs, sem):
            r1, r2, r3 = bufs
            pltpu.async_copy(r1, acc.at[idx_v.at[ioff + k]], sem, add=True)

        def drain_scat(bufs, sem):
            r1, r2, r3 = bufs
            pltpu.make_async_copy(r1, acc.at[pl.ds(0, _C)], sem).wait()

        def zrow_body(r, carry):
            for j in range(_D // 16):
                r1a[r, pl.ds(j * 16, 16)] = jnp.zeros((16,), jnp.float32)
            return carry

        lax.fori_loop(0, _RPT, zrow_body, 0)
        pltpu.sync_copy(r1a.at[pl.ds(0, _RPT)],
                        acc.at[pl.ds(sid * _RPT, _RPT)])
        pltpu.sync_copy(sclh, scl_v)

        s_w = wid * _CPW + jnp.minimum(wid, _XTRA)
        abase = s_w // 8 * 8
        ioff = s_w - abase
        pltpu.sync_copy(bh.at[pl.ds(abase, _IPW)], idx_v)
        plsc.subcore_barrier()

        issue_loads(s_w, bufs_a, lsa)

        def pair_body(p, carry):
            c0 = s_w + 2 * p

            @pl.when(p > 0)
            def _():
                drain_scat(bufs_b, ssb)

            issue_loads(c0 + 1, bufs_b, lsb)
            drain_loads(bufs_a, lsa)
            combine(bufs_a)
            issue_scat(2 * p, bufs_a, ssa)

            @pl.when(p < _NPAIR - 1)
            def _():
                drain_scat(bufs_a, ssa)
                issue_loads(c0 + 2, bufs_a, lsa)

            drain_loads(bufs_b, lsb)
            combine(bufs_b)
            issue_scat(2 * p + 1, bufs_b, ssb)
            return carry

        lax.fori_loop(0, _NPAIR, pair_body, 0)
        drain_scat(bufs_a, ssa)
        drain_scat(bufs_b, ssb)

        @pl.when(wid < _XTRA)
        def _():
            issue_loads(s_w + _CPW, bufs_a, lsa)
            drain_loads(bufs_a, lsa)
            combine(bufs_a)
            issue_scat(_CPW, bufs_a, ssa)
            drain_scat(bufs_a, ssa)

        plsc.subcore_barrier()

        r0 = sid * _RPT
        pltpu.sync_copy(acc.at[pl.ds(r0, _RPT)], r1a.at[pl.ds(0, _RPT)])
        pltpu.sync_copy(r1a.at[pl.ds(0, _RPT)],
                        outh.at[cid].at[pl.ds(r0, _RPT)])

    return body(x1, x2, x3, batch, scalesb)


def _tc_add(partials):
    def body(p_ref, o_ref):
        o_ref[...] = p_ref[0] + p_ref[1]

    return pl.pallas_call(
        body,
        out_shape=jax.ShapeDtypeStruct((_S, _D), jnp.float32),
    )(partials)


def kernel(x1, x2, x3, batch, scales):
    batch_i = batch.astype(jnp.int32)
    batch2d = jnp.pad(batch_i.reshape(_NCHUNK, _C), ((0, _IPW), (0, 0)))
    scalesb = jnp.broadcast_to(
        scales.astype(jnp.float32)[:, None], (3, 16))
    partials = _sc_segment_sum(x1, x2, x3, batch2d, scalesb)
    return _tc_add(partials)


# speedup vs baseline: 1.3553x; 1.0034x over previous
import functools

import jax
import jax.numpy as jnp
from jax import lax
from jax.experimental import pallas as pl
from jax.experimental.pallas import tpu as pltpu
from jax.experimental.pallas import tpu_sc as plsc

_N = 320000
_D = 128
_S = 1024
_C = 128
_NCHUNK = _N // _C
_NC = 2
_NS = 16
_NW = _NC * _NS
_CPW = _NCHUNK // _NW
_XTRA = _NCHUNK - _CPW * _NW
_NPAIR = _CPW // 2
_IPW = _CPW + 1 + 9
_RPT = _S // _NS


def _sc_segment_sum(x1, x2, x3, batch, scalesb):
    mesh = plsc.VectorSubcoreMesh(core_axis_name="c", subcore_axis_name="s")

    @functools.partial(
        pl.kernel,
        mesh=mesh,
        out_type=jax.ShapeDtypeStruct((_NC, _S, _D), jnp.float32),
        scratch_types=[
            pltpu.VMEM((_C, _D), jnp.float32),
            pltpu.VMEM((_C, _D), jnp.float32),
            pltpu.VMEM((_C, _D), jnp.float32),
            pltpu.VMEM((_C, _D), jnp.float32),
            pltpu.VMEM((_C, _D), jnp.float32),
            pltpu.VMEM((_C, _D), jnp.float32),
            pltpu.VMEM((_IPW, _C), jnp.int32),
            pltpu.VMEM((3, 16), jnp.float32),
            pltpu.VMEM_SHARED((_S, _D), jnp.float32),
            pltpu.VMEM_SHARED((_S, _D), jnp.float32),
            pltpu.SemaphoreType.DMA,
            pltpu.SemaphoreType.DMA,
            pltpu.SemaphoreType.DMA,
            pltpu.SemaphoreType.DMA,
        ],
    )
    def body(x1h, x2h, x3h, bh, sclh, outh,
             r1a, r2a, r3a, r1b, r2b, r3b, idx_v, scl_v,
             acc, acc2, lsa, lsb, ssa, ssb):
        cid = lax.axis_index("c")
        sid = lax.axis_index("s")
        wid = sid * _NC + cid
        bufs_a = (r1a, r2a, r3a)
        bufs_b = (r1b, r2b, r3b)

        def issue_loads(c, bufs, sem):
            base = c * _C
            r1, r2, r3 = bufs
            pltpu.async_copy(x1h.at[pl.ds(base, _C)], r1, sem)
            pltpu.async_copy(x2h.at[pl.ds(base, _C)], r2, sem)
            pltpu.async_copy(x3h.at[pl.ds(base, _C)], r3, sem)

        def drain_loads(bufs, sem):
            r1, r2, r3 = bufs
            pltpu.make_async_copy(x1h.at[pl.ds(0, _C)], r1, sem).wait()
            pltpu.make_async_copy(x2h.at[pl.ds(0, _C)], r2, sem).wait()
            pltpu.make_async_copy(x3h.at[pl.ds(0, _C)], r3, sem).wait()

        def combine(bufs):
            r1, r2, r3 = bufs
            s1 = scl_v[0]
            s2 = scl_v[1]
            s3 = scl_v[2]

            def row_body(r, carry):
                for j in range(_D // 16):
                    sl = pl.ds(j * 16, 16)
                    r1[r, sl] = (r1[r, sl] * s1 + r2[r, sl] * s2
                                 + r3[r, sl] * s3)
                return carry

            lax.fori_loop(0, _C, row_body, 0)

        def issue_scat(k, bufs, sem):
            r1, r2, r3 = bufs

            @pl.when(sid < _NS // 2)
            def _():
                pltpu.async_copy(r1, acc.at[idx_v.at[ioff + k]], sem,
                                 add=True)

            @pl.when(sid >= _NS // 2)
            def _():
                pltpu.async_copy(r1, acc2.at[idx_v.at[ioff + k]], sem,
                                 add=True)

        def drain_scat(bufs, sem):
            r1, r2, r3 = bufs
            pltpu.make_async_copy(r1, acc.at[pl.ds(0, _C)], sem).wait()

        def zrow_body(r, carry):
            for j in range(_D // 16):
                r1a[r, pl.ds(j * 16, 16)] = jnp.zeros((16,), jnp.float32)
            return carry

        lax.fori_loop(0, _RPT, zrow_body, 0)
        pltpu.sync_copy(r1a.at[pl.ds(0, _RPT)],
                        acc.at[pl.ds(sid * _RPT, _RPT)])
        pltpu.sync_copy(r1a.at[pl.ds(0, _RPT)],
                        acc2.at[pl.ds(sid * _RPT, _RPT)])
        pltpu.sync_copy(sclh, scl_v)

        s_w = wid * _CPW + jnp.minimum(wid, _XTRA)
        abase = s_w // 8 * 8
        ioff = s_w - abase
        pltpu.sync_copy(bh.at[pl.ds(abase, _IPW)], idx_v)
        plsc.subcore_barrier()

        issue_loads(s_w, bufs_a, lsa)

        def pair_body(p, carry):
            c0 = s_w + 2 * p

            @pl.when(p > 0)
            def _():
                drain_scat(bufs_b, ssb)

            issue_loads(c0 + 1, bufs_b, lsb)
            drain_loads(bufs_a, lsa)
            combine(bufs_a)
            issue_scat(2 * p, bufs_a, ssa)

            @pl.when(p < _NPAIR - 1)
            def _():
                drain_scat(bufs_a, ssa)
                issue_loads(c0 + 2, bufs_a, lsa)

            drain_loads(bufs_b, lsb)
            combine(bufs_b)
            issue_scat(2 * p + 1, bufs_b, ssb)
            return carry

        lax.fori_loop(0, _NPAIR, pair_body, 0)
        drain_scat(bufs_a, ssa)
        drain_scat(bufs_b, ssb)

        @pl.when(wid < _XTRA)
        def _():
            issue_loads(s_w + _CPW, bufs_a, lsa)
            drain_loads(bufs_a, lsa)
            combine(bufs_a)
            issue_scat(_CPW, bufs_a, ssa)
            drain_scat(bufs_a, ssa)

        plsc.subcore_barrier()

        r0 = sid * _RPT
        pltpu.sync_copy(acc.at[pl.ds(r0, _RPT)], r1a.at[pl.ds(0, _RPT)])
        pltpu.sync_copy(acc2.at[pl.ds(r0, _RPT)], r2a.at[pl.ds(0, _RPT)])

        def sum_body(r, carry):
            for j in range(_D // 16):
                sl = pl.ds(j * 16, 16)
                r1a[r, sl] = r1a[r, sl] + r2a[r, sl]
            return carry

        lax.fori_loop(0, _RPT, sum_body, 0)
        pltpu.sync_copy(r1a.at[pl.ds(0, _RPT)],
                        outh.at[cid].at[pl.ds(r0, _RPT)])

    return body(x1, x2, x3, batch, scalesb)


def _tc_add(partials):
    def body(p_ref, o_ref):
        o_ref[...] = p_ref[0] + p_ref[1]

    return pl.pallas_call(
        body,
        out_shape=jax.ShapeDtypeStruct((_S, _D), jnp.float32),
    )(partials)


def kernel(x1, x2, x3, batch, scales):
    batch_i = batch.astype(jnp.int32)
    batch2d = jnp.pad(batch_i.reshape(_NCHUNK, _C), ((0, _IPW), (0, 0)))
    scalesb = jnp.broadcast_to(
        scales.astype(jnp.float32)[:, None], (3, 16))
    partials = _sc_segment_sum(x1, x2, x3, batch2d, scalesb)
    return _tc_add(partials)


# speedup vs baseline: 1.4488x; 1.0689x over previous
import functools

import jax
import jax.numpy as jnp
from jax import lax
from jax.experimental import pallas as pl
from jax.experimental.pallas import tpu as pltpu
from jax.experimental.pallas import tpu_sc as plsc

_N = 320000
_D = 128
_S = 1024
_C = 128
_NCHUNK = _N // _C
_NC = 2
_NS = 16
_NW = _NC * _NS
_CPW = _NCHUNK // _NW
_XTRA = _NCHUNK - _CPW * _NW
_NPAIR = _CPW // 2
_IPW = _CPW + 1 + 9
_RPT = _S // _NS


def _sc_segment_sum(x1, x2, x3, batch, scalesb):
    mesh = plsc.VectorSubcoreMesh(core_axis_name="c", subcore_axis_name="s")

    @functools.partial(
        pl.kernel,
        mesh=mesh,
        out_type=jax.ShapeDtypeStruct((_NC, _S, _D), jnp.float32),
        scratch_types=[
            pltpu.VMEM((_C, _D), jnp.float32),
            pltpu.VMEM((_C, _D), jnp.float32),
            pltpu.VMEM((_C, _D), jnp.float32),
            pltpu.VMEM((_C, _D), jnp.float32),
            pltpu.VMEM((_C, _D), jnp.float32),
            pltpu.VMEM((_C, _D), jnp.float32),
            pltpu.VMEM((_IPW, _C), jnp.int32),
            pltpu.VMEM((3, 16), jnp.float32),
            pltpu.VMEM_SHARED((_S, _D), jnp.float32),
            pltpu.SemaphoreType.DMA,
            pltpu.SemaphoreType.DMA,
            pltpu.SemaphoreType.DMA,
            pltpu.SemaphoreType.DMA,
        ],
    )
    def body(x1h, x2h, x3h, bh, sclh, outh,
             r1a, r2a, r3a, r1b, r2b, r3b, idx_v, scl_v,
             acc, lsa, lsb, ssa, ssb):
        cid = lax.axis_index("c")
        sid = lax.axis_index("s")
        wid = sid * _NC + cid
        bufs_a = (r1a, r2a, r3a)
        bufs_b = (r1b, r2b, r3b)

        def issue_loads23(c, bufs, sem):
            base = c * _C
            r1, r2, r3 = bufs
            pltpu.async_copy(x2h.at[pl.ds(base, _C)], r2, sem)
            pltpu.async_copy(x3h.at[pl.ds(base, _C)], r3, sem)

        def issue_load1(c, bufs, sem):
            base = c * _C
            r1, r2, r3 = bufs
            pltpu.async_copy(x1h.at[pl.ds(base, _C)], r1, sem)

        def issue_loads(c, bufs, sem):
            issue_loads23(c, bufs, sem)
            issue_load1(c, bufs, sem)

        def drain_loads(bufs, sem):
            r1, r2, r3 = bufs
            pltpu.make_async_copy(x1h.at[pl.ds(0, _C)], r1, sem).wait()
            pltpu.make_async_copy(x2h.at[pl.ds(0, _C)], r2, sem).wait()
            pltpu.make_async_copy(x3h.at[pl.ds(0, _C)], r3, sem).wait()

        def combine(bufs):
            r1, r2, r3 = bufs
            s1 = scl_v[0]
            s2 = scl_v[1]
            s3 = scl_v[2]

            def row_body(r, carry):
                for j in range(_D // 16):
                    sl = pl.ds(j * 16, 16)
                    r1[r, sl] = (r1[r, sl] * s1 + r2[r, sl] * s2
                                 + r3[r, sl] * s3)
                return carry

            lax.fori_loop(0, _C, row_body, 0)

        def issue_scat(k, bufs, sem):
            r1, r2, r3 = bufs
            pltpu.async_copy(r1, acc.at[idx_v.at[ioff + k]], sem, add=True)

        def drain_scat(bufs, sem):
            r1, r2, r3 = bufs
            pltpu.make_async_copy(r1, acc.at[pl.ds(0, _C)], sem).wait()

        def zrow_body(r, carry):
            for j in range(_D // 16):
                r1a[r, pl.ds(j * 16, 16)] = jnp.zeros((16,), jnp.float32)
            return carry

        lax.fori_loop(0, _RPT, zrow_body, 0)
        pltpu.sync_copy(r1a.at[pl.ds(0, _RPT)],
                        acc.at[pl.ds(sid * _RPT, _RPT)])
        pltpu.sync_copy(sclh, scl_v)

        s_w = wid * _CPW + jnp.minimum(wid, _XTRA)
        abase = s_w // 8 * 8
        ioff = s_w - abase
        pltpu.sync_copy(bh.at[pl.ds(abase, _IPW)], idx_v)
        plsc.subcore_barrier()

        issue_loads(s_w, bufs_a, lsa)

        def pair_body(p, carry):
            c0 = s_w + 2 * p

            issue_loads23(c0 + 1, bufs_b, lsb)

            @pl.when(p > 0)
            def _():
                drain_scat(bufs_b, ssb)

            issue_load1(c0 + 1, bufs_b, lsb)
            drain_loads(bufs_a, lsa)
            combine(bufs_a)
            issue_scat(2 * p, bufs_a, ssa)

            @pl.when(p < _NPAIR - 1)
            def _():
                issue_loads23(c0 + 2, bufs_a, lsa)
                drain_scat(bufs_a, ssa)
                issue_load1(c0 + 2, bufs_a, lsa)

            drain_loads(bufs_b, lsb)
            combine(bufs_b)
            issue_scat(2 * p + 1, bufs_b, ssb)
            return carry

        lax.fori_loop(0, _NPAIR, pair_body, 0)
        drain_scat(bufs_a, ssa)
        drain_scat(bufs_b, ssb)

        @pl.when(wid < _XTRA)
        def _():
            issue_loads(s_w + _CPW, bufs_a, lsa)
            drain_loads(bufs_a, lsa)
            combine(bufs_a)
            issue_scat(_CPW, bufs_a, ssa)
            drain_scat(bufs_a, ssa)

        plsc.subcore_barrier()

        r0 = sid * _RPT
        pltpu.sync_copy(acc.at[pl.ds(r0, _RPT)], r1a.at[pl.ds(0, _RPT)])
        pltpu.sync_copy(r1a.at[pl.ds(0, _RPT)],
                        outh.at[cid].at[pl.ds(r0, _RPT)])

    return body(x1, x2, x3, batch, scalesb)


def _tc_add(partials):
    def body(p_ref, o_ref):
        o_ref[...] = p_ref[0] + p_ref[1]

    return pl.pallas_call(
        body,
        out_shape=jax.ShapeDtypeStruct((_S, _D), jnp.float32),
    )(partials)


def kernel(x1, x2, x3, batch, scales):
    batch_i = batch.astype(jnp.int32)
    batch2d = jnp.pad(batch_i.reshape(_NCHUNK, _C), ((0, _IPW), (0, 0)))
    scalesb = jnp.broadcast_to(
        scales.astype(jnp.float32)[:, None], (3, 16))
    partials = _sc_segment_sum(x1, x2, x3, batch2d, scalesb)
    return _tc_add(partials)
